# Initial kernel scaffold; baseline (speedup 1.0000x reference)
import functools

import jax
import jax.numpy as jnp
from jax import lax
from jax.experimental import pallas as pl
from jax.experimental.pallas import tpu as pltpu
from jax.experimental.pallas import tpu_sc as plsc

N_NODES = 10000
IN_DIM = 128
HID = 64

NPAD = 10240
NC, NS = 2, 16
NW = NC * NS
EPW = 10240
EPAD = NW * EPW
CHUNK = 1024
GROUPS = 8
NCHUNK = EPW // CHUNK
ROWS_PER_W = EPW // 128
ROWS_PER_CHUNK = CHUNK // 128
TILE_ROWS = NPAD // NS
DEAD_DST = 10100

_sc_mesh = plsc.VectorSubcoreMesh(core_axis_name="c", subcore_axis_name="s")


@functools.partial(
    pl.kernel,
    out_type=jax.ShapeDtypeStruct((NC, NPAD, HID), jnp.float32),
    mesh=_sc_mesh,
    scratch_types=[
        pltpu.VMEM((ROWS_PER_CHUNK, 128), jnp.int32),
        pltpu.VMEM((ROWS_PER_CHUNK, 128), jnp.int32),
        pltpu.VMEM((CHUNK, HID), jnp.float32),
        pltpu.VMEM_SHARED((NPAD, HID), jnp.float32),
        pltpu.SemaphoreType.DMA,
    ],
)
def _sc_propagate(g_hbm, src_hbm, dst_hbm, zeros_hbm, out_hbm,
                  sidx, didx, rows, acc_sh, sem):
    c = lax.axis_index("c")
    s = lax.axis_index("s")
    w = s * NC + c

    pltpu.sync_copy(zeros_hbm, rows.at[pl.ds(0, TILE_ROWS)])
    pltpu.sync_copy(rows.at[pl.ds(0, TILE_ROWS)],
                    acc_sh.at[pl.ds(s * TILE_ROWS, TILE_ROWS)])
    plsc.subcore_barrier()

    @pl.loop(0, NCHUNK)
    def _chunk(ci):
        row0 = w * ROWS_PER_W + ci * ROWS_PER_CHUNK
        pltpu.sync_copy(src_hbm.at[pl.ds(row0, ROWS_PER_CHUNK)], sidx)
        pltpu.sync_copy(dst_hbm.at[pl.ds(row0, ROWS_PER_CHUNK)], didx)
        copies = [
            pltpu.async_copy(g_hbm.at[sidx.at[j]],
                             rows.at[pl.ds(j * 128, 128)], sem)
            for j in range(GROUPS)
        ]
        for d in copies:
            d.wait()
        for j in range(GROUPS):
            pltpu.sync_copy(rows.at[pl.ds(j * 128, 128)],
                            acc_sh.at[didx.at[j]], add=True)

    plsc.subcore_barrier()
    pltpu.sync_copy(acc_sh.at[pl.ds(s * TILE_ROWS, TILE_ROWS)],
                    rows.at[pl.ds(0, TILE_ROWS)])
    pltpu.sync_copy(rows.at[pl.ds(0, TILE_ROWS)],
                    out_hbm.at[c].at[pl.ds(s * TILE_ROWS, TILE_ROWS)])


def _tc_matmul_kernel(x_ref, w_ref, o_ref):
    o_ref[...] = jnp.dot(x_ref[...], w_ref[...],
                         preferred_element_type=jnp.float32)


def _tc_prep_kernel(deg_ref, h_ref, dinv_ref, g_ref):
    deg = deg_ref[0, :, 0:1] + deg_ref[1, :, 0:1] + 1.0
    dinv = lax.rsqrt(deg)
    dinv_ref[...] = dinv
    g_ref[...] = h_ref[...] * dinv


def _tc_layer_kernel(acc_ref, g_ref, dinv_ref, b_ref, w_ref, o_ref):
    dinv = dinv_ref[...]
    t = dinv * (acc_ref[0] + acc_ref[1] + g_ref[...]) + b_ref[...]
    t = jnp.maximum(t, 0.0)
    o_ref[...] = dinv * jnp.dot(t, w_ref[...],
                                preferred_element_type=jnp.float32)


def _tc_final_kernel(acc_ref, g_ref, dinv_ref, b_ref, o_ref):
    o_ref[...] = (dinv_ref[...] * (acc_ref[0] + acc_ref[1] + g_ref[...])
                  + b_ref[...])


def _tc_call(body, out_shapes):
    return pl.pallas_call(body, out_shape=out_shapes)


def kernel(x, edge_index, W1, b1, W2, b2, W3, b3):
    src = edge_index[0]
    dst = edge_index[1]
    e = src.shape[0]
    fill = EPAD - e
    src2d = jnp.concatenate(
        [src, jnp.zeros((fill,), jnp.int32)]).reshape(EPAD // 128, 128)
    dst2d = jnp.concatenate(
        [dst, jnp.full((fill,), DEAD_DST, jnp.int32)]).reshape(EPAD // 128, 128)
    x_pad = jnp.concatenate(
        [x, jnp.zeros((NPAD - N_NODES, IN_DIM), jnp.float32)])
    zeros_tile = jnp.zeros((TILE_ROWS, HID), jnp.float32)
    ones_mat = jnp.ones((NPAD, HID), jnp.float32)
    b1r = b1.reshape(1, HID)
    b2r = b2.reshape(1, HID)
    b3r = b3.reshape(1, HID)

    deg = _sc_propagate(ones_mat, src2d, dst2d, zeros_tile)

    h1 = _tc_call(_tc_matmul_kernel,
                  jax.ShapeDtypeStruct((NPAD, HID), jnp.float32))(x_pad, W1)
    dinv, g1 = _tc_call(
        _tc_prep_kernel,
        (jax.ShapeDtypeStruct((NPAD, 1), jnp.float32),
         jax.ShapeDtypeStruct((NPAD, HID), jnp.float32)))(deg, h1)

    acc1 = _sc_propagate(g1, src2d, dst2d, zeros_tile)
    g2 = _tc_call(_tc_layer_kernel,
                  jax.ShapeDtypeStruct((NPAD, HID), jnp.float32))(
                      acc1, g1, dinv, b1r, W2)
    acc2 = _sc_propagate(g2, src2d, dst2d, zeros_tile)
    g3 = _tc_call(_tc_layer_kernel,
                  jax.ShapeDtypeStruct((NPAD, HID), jnp.float32))(
                      acc2, g2, dinv, b2r, W3)
    acc3 = _sc_propagate(g3, src2d, dst2d, zeros_tile)
    out = _tc_call(_tc_final_kernel,
                   jax.ShapeDtypeStruct((NPAD, HID), jnp.float32))(
                       acc3, g3, dinv, b3r)
    return out[:N_NODES]


# speedup vs baseline: 9.1707x; 9.1707x over previous
import functools

import jax
import jax.numpy as jnp
from jax import lax
from jax.experimental import pallas as pl
from jax.experimental.pallas import tpu as pltpu
from jax.experimental.pallas import tpu_sc as plsc

N_NODES = 10000
IN_DIM = 128
HID = 64

NPAD = 10240
NC, NS = 2, 16
NW = NC * NS
EPW = 10240
EPAD = NW * EPW
CHUNK = 1024
GROUPS = 8
NCHUNK = EPW // CHUNK
ROWS_PER_W = EPW // 128
ROWS_PER_CHUNK = CHUNK // 128
TILE_ROWS = NPAD // NS
DEAD_DST = 10100

_sc_mesh = plsc.VectorSubcoreMesh(core_axis_name="c", subcore_axis_name="s")


@functools.partial(
    pl.kernel,
    out_type=jax.ShapeDtypeStruct((NC, NPAD, HID), jnp.float32),
    mesh=_sc_mesh,
    scratch_types=[
        pltpu.VMEM((ROWS_PER_CHUNK, 128), jnp.int32),
        pltpu.VMEM((ROWS_PER_CHUNK, 128), jnp.int32),
        pltpu.VMEM((CHUNK, HID), jnp.float32),
        pltpu.VMEM_SHARED((NPAD, HID), jnp.float32),
        pltpu.SemaphoreType.DMA,
    ],
    compiler_params=pltpu.CompilerParams(use_tc_tiling_on_sc=False),
)
def _sc_propagate(g_hbm, src_hbm, dst_hbm, zeros_hbm, out_hbm,
                  sidx, didx, rows, acc_sh, sem):
    c = lax.axis_index("c")
    s = lax.axis_index("s")
    w = s * NC + c

    pltpu.sync_copy(zeros_hbm, rows.at[pl.ds(0, TILE_ROWS)])
    pltpu.sync_copy(rows.at[pl.ds(0, TILE_ROWS)],
                    acc_sh.at[pl.ds(s * TILE_ROWS, TILE_ROWS)])
    plsc.subcore_barrier()

    @pl.loop(0, NCHUNK)
    def _chunk(ci):
        row0 = w * ROWS_PER_W + ci * ROWS_PER_CHUNK
        pltpu.sync_copy(src_hbm.at[pl.ds(row0, ROWS_PER_CHUNK)], sidx)
        pltpu.sync_copy(dst_hbm.at[pl.ds(row0, ROWS_PER_CHUNK)], didx)
        copies = [
            pltpu.async_copy(g_hbm.at[sidx.at[j]],
                             rows.at[pl.ds(j * 128, 128)], sem)
            for j in range(GROUPS)
        ]
        for d in copies:
            d.wait()
        for j in range(GROUPS):
            pltpu.sync_copy(rows.at[pl.ds(j * 128, 128)],
                            acc_sh.at[didx.at[j]], add=True)

    plsc.subcore_barrier()
    pltpu.sync_copy(acc_sh.at[pl.ds(s * TILE_ROWS, TILE_ROWS)],
                    rows.at[pl.ds(0, TILE_ROWS)])
    pltpu.sync_copy(rows.at[pl.ds(0, TILE_ROWS)],
                    out_hbm.at[c].at[pl.ds(s * TILE_ROWS, TILE_ROWS)])


def _tc_matmul_kernel(x_ref, w_ref, o_ref):
    o_ref[...] = jnp.dot(x_ref[...], w_ref[...],
                         preferred_element_type=jnp.float32)


def _tc_prep_kernel(deg_ref, h_ref, dinv_ref, g_ref):
    deg = deg_ref[0, :, 0:1] + deg_ref[1, :, 0:1] + 1.0
    dinv = lax.rsqrt(deg)
    dinv_ref[...] = dinv
    g_ref[...] = h_ref[...] * dinv


def _tc_layer_kernel(acc_ref, g_ref, dinv_ref, b_ref, w_ref, o_ref):
    dinv = dinv_ref[...]
    t = dinv * (acc_ref[0] + acc_ref[1] + g_ref[...]) + b_ref[...]
    t = jnp.maximum(t, 0.0)
    o_ref[...] = dinv * jnp.dot(t, w_ref[...],
                                preferred_element_type=jnp.float32)


def _tc_final_kernel(acc_ref, g_ref, dinv_ref, b_ref, o_ref):
    o_ref[...] = (dinv_ref[...] * (acc_ref[0] + acc_ref[1] + g_ref[...])
                  + b_ref[...])


def _tc_call(body, out_shapes):
    return pl.pallas_call(body, out_shape=out_shapes)


def kernel(x, edge_index, W1, b1, W2, b2, W3, b3):
    src = edge_index[0]
    dst = edge_index[1]
    e = src.shape[0]
    fill = EPAD - e
    src2d = jnp.concatenate(
        [src, jnp.zeros((fill,), jnp.int32)]).reshape(EPAD // 128, 128)
    dst2d = jnp.concatenate(
        [dst, jnp.full((fill,), DEAD_DST, jnp.int32)]).reshape(EPAD // 128, 128)
    x_pad = jnp.concatenate(
        [x, jnp.zeros((NPAD - N_NODES, IN_DIM), jnp.float32)])
    zeros_tile = jnp.zeros((TILE_ROWS, HID), jnp.float32)
    ones_mat = jnp.ones((NPAD, HID), jnp.float32)
    b1r = b1.reshape(1, HID)
    b2r = b2.reshape(1, HID)
    b3r = b3.reshape(1, HID)

    deg = _sc_propagate(ones_mat, src2d, dst2d, zeros_tile)

    h1 = _tc_call(_tc_matmul_kernel,
                  jax.ShapeDtypeStruct((NPAD, HID), jnp.float32))(x_pad, W1)
    dinv, g1 = _tc_call(
        _tc_prep_kernel,
        (jax.ShapeDtypeStruct((NPAD, 1), jnp.float32),
         jax.ShapeDtypeStruct((NPAD, HID), jnp.float32)))(deg, h1)

    acc1 = _sc_propagate(g1, src2d, dst2d, zeros_tile)
    g2 = _tc_call(_tc_layer_kernel,
                  jax.ShapeDtypeStruct((NPAD, HID), jnp.float32))(
                      acc1, g1, dinv, b1r, W2)
    acc2 = _sc_propagate(g2, src2d, dst2d, zeros_tile)
    g3 = _tc_call(_tc_layer_kernel,
                  jax.ShapeDtypeStruct((NPAD, HID), jnp.float32))(
                      acc2, g2, dinv, b2r, W3)
    acc3 = _sc_propagate(g3, src2d, dst2d, zeros_tile)
    out = _tc_call(_tc_final_kernel,
                   jax.ShapeDtypeStruct((NPAD, HID), jnp.float32))(
                       acc3, g3, dinv, b3r)
    return out[:N_NODES]


# speedup vs baseline: 14.5615x; 1.5878x over previous
import functools

import jax
import jax.numpy as jnp
from jax import lax
from jax.experimental import pallas as pl
from jax.experimental.pallas import tpu as pltpu
from jax.experimental.pallas import tpu_sc as plsc

N_NODES = 10000
IN_DIM = 128
HID = 64

NPAD = 10240
NC, NS = 2, 16
NW = NC * NS
EPW = 10240
EPAD = NW * EPW
CHUNK = 512
GROUPS = CHUNK // 128
NCHUNK = EPW // CHUNK
ROWS_PER_W = EPW // 128
ROWS_PER_CHUNK = CHUNK // 128
TILE_ROWS = NPAD // NS
DEAD_DST = 10100
DEGW = 16

_sc_mesh = plsc.VectorSubcoreMesh(core_axis_name="c", subcore_axis_name="s")


_HALF = TILE_ROWS // 2


@functools.partial(
    pl.kernel,
    out_type=jax.ShapeDtypeStruct((NC, NPAD, HID), jnp.float32),
    mesh=_sc_mesh,
    scratch_types=[
        pltpu.VMEM((ROWS_PER_W, 128), jnp.int32),
        pltpu.VMEM((ROWS_PER_W, 128), jnp.int32),
        pltpu.VMEM((CHUNK, HID), jnp.float32),
        pltpu.VMEM((CHUNK, HID), jnp.float32),
        pltpu.VMEM_SHARED((NPAD, HID), jnp.float32),
        pltpu.SemaphoreType.DMA,
        pltpu.SemaphoreType.DMA,
        pltpu.SemaphoreType.DMA,
        pltpu.SemaphoreType.DMA,
    ],
    compiler_params=pltpu.CompilerParams(use_tc_tiling_on_sc=False),
)
def _sc_propagate(g_hbm, src_hbm, dst_hbm, zeros_hbm, out_hbm,
                  sidx, didx, rows_a, rows_b, acc_sh,
                  sem_ga, sem_gb, sem_sa, sem_sb):
    c = lax.axis_index("c")
    s = lax.axis_index("s")
    w = s * NC + c

    pltpu.sync_copy(src_hbm.at[pl.ds(w * ROWS_PER_W, ROWS_PER_W)], sidx)
    pltpu.sync_copy(dst_hbm.at[pl.ds(w * ROWS_PER_W, ROWS_PER_W)], didx)

    pltpu.sync_copy(zeros_hbm, rows_a.at[pl.ds(0, _HALF)])
    for h in range(2):
        pltpu.sync_copy(rows_a.at[pl.ds(0, _HALF)],
                        acc_sh.at[pl.ds(s * TILE_ROWS + h * _HALF, _HALF)])
    plsc.subcore_barrier()

    @pl.loop(0, NCHUNK // 2)
    def _pair(i):
        r0 = i * 2 * GROUPS
        ga = [pltpu.async_copy(g_hbm.at[sidx.at[r0 + j]],
                               rows_a.at[pl.ds(j * 128, 128)], sem_ga)
              for j in range(GROUPS)]
        gb = [pltpu.async_copy(g_hbm.at[sidx.at[r0 + GROUPS + j]],
                               rows_b.at[pl.ds(j * 128, 128)], sem_gb)
              for j in range(GROUPS)]
        for d in ga:
            d.wait()
        sa = [pltpu.async_copy(rows_a.at[pl.ds(j * 128, 128)],
                               acc_sh.at[didx.at[r0 + j]], sem_sa, add=True)
              for j in range(GROUPS)]
        for d in gb:
            d.wait()
        sb = [pltpu.async_copy(rows_b.at[pl.ds(j * 128, 128)],
                               acc_sh.at[didx.at[r0 + GROUPS + j]], sem_sb,
                               add=True)
              for j in range(GROUPS)]
        for d in sa + sb:
            d.wait()

    plsc.subcore_barrier()
    for h in range(2):
        pltpu.sync_copy(acc_sh.at[pl.ds(s * TILE_ROWS + h * _HALF, _HALF)],
                        rows_a.at[pl.ds(0, _HALF)])
        pltpu.sync_copy(rows_a.at[pl.ds(0, _HALF)],
                        out_hbm.at[c].at[pl.ds(s * TILE_ROWS + h * _HALF,
                                               _HALF)])


@functools.partial(
    pl.kernel,
    out_type=jax.ShapeDtypeStruct((NC, NPAD, DEGW), jnp.float32),
    mesh=_sc_mesh,
    scratch_types=[
        pltpu.VMEM((ROWS_PER_W, 128), jnp.int32),
        pltpu.VMEM((128, DEGW), jnp.float32),
        pltpu.VMEM((TILE_ROWS, DEGW), jnp.float32),
        pltpu.VMEM_SHARED((NPAD, DEGW), jnp.float32),
        pltpu.SemaphoreType.DMA,
    ],
    compiler_params=pltpu.CompilerParams(use_tc_tiling_on_sc=False),
)
def _sc_degree(dst_hbm, ones_hbm, zeros_hbm, out_hbm,
               didx, ones_v, stage, deg_sh, sem):
    c = lax.axis_index("c")
    s = lax.axis_index("s")
    w = s * NC + c

    pltpu.sync_copy(dst_hbm.at[pl.ds(w * ROWS_PER_W, ROWS_PER_W)], didx)
    pltpu.sync_copy(ones_hbm, ones_v)
    pltpu.sync_copy(zeros_hbm, stage)
    pltpu.sync_copy(stage, deg_sh.at[pl.ds(s * TILE_ROWS, TILE_ROWS)])
    plsc.subcore_barrier()

    @pl.loop(0, NCHUNK // 2)
    def _pair(i):
        r0 = i * 2 * GROUPS
        ds_ = [pltpu.async_copy(ones_v, deg_sh.at[didx.at[r0 + j]], sem,
                                add=True)
               for j in range(2 * GROUPS)]
        for d in ds_:
            d.wait()

    plsc.subcore_barrier()
    pltpu.sync_copy(deg_sh.at[pl.ds(s * TILE_ROWS, TILE_ROWS)], stage)
    pltpu.sync_copy(stage, out_hbm.at[c].at[pl.ds(s * TILE_ROWS, TILE_ROWS)])


def _tc_matmul_kernel(x_ref, w_ref, o_ref):
    o_ref[...] = jnp.dot(x_ref[...], w_ref[...],
                         preferred_element_type=jnp.float32)


def _tc_prep_kernel(deg_ref, h_ref, dinv_ref, g_ref):
    deg = deg_ref[0, :, 0:1] + deg_ref[1, :, 0:1] + 1.0
    dinv = lax.rsqrt(deg)
    dinv_ref[...] = dinv
    g_ref[...] = h_ref[...] * dinv


def _tc_layer_kernel(acc_ref, g_ref, dinv_ref, b_ref, w_ref, o_ref):
    dinv = dinv_ref[...]
    t = dinv * (acc_ref[0] + acc_ref[1] + g_ref[...]) + b_ref[...]
    t = jnp.maximum(t, 0.0)
    o_ref[...] = dinv * jnp.dot(t, w_ref[...],
                                preferred_element_type=jnp.float32)


def _tc_final_kernel(acc_ref, g_ref, dinv_ref, b_ref, o_ref):
    o_ref[...] = (dinv_ref[...] * (acc_ref[0] + acc_ref[1] + g_ref[...])
                  + b_ref[...])


def _tc_call(body, out_shapes):
    return pl.pallas_call(body, out_shape=out_shapes)


def kernel(x, edge_index, W1, b1, W2, b2, W3, b3):
    src = edge_index[0]
    dst = edge_index[1]
    e = src.shape[0]
    fill = EPAD - e
    src2d = jnp.concatenate(
        [src, jnp.zeros((fill,), jnp.int32)]).reshape(EPAD // 128, 128)
    dst2d = jnp.concatenate(
        [dst, jnp.full((fill,), DEAD_DST, jnp.int32)]).reshape(EPAD // 128, 128)
    x_pad = jnp.concatenate(
        [x, jnp.zeros((NPAD - N_NODES, IN_DIM), jnp.float32)])
    zeros_tile = jnp.zeros((_HALF, HID), jnp.float32)
    ones_deg = jnp.ones((128, DEGW), jnp.float32)
    zeros_deg = jnp.zeros((TILE_ROWS, DEGW), jnp.float32)
    b1r = b1.reshape(1, HID)
    b2r = b2.reshape(1, HID)
    b3r = b3.reshape(1, HID)

    deg = _sc_degree(dst2d, ones_deg, zeros_deg)

    h1 = _tc_call(_tc_matmul_kernel,
                  jax.ShapeDtypeStruct((NPAD, HID), jnp.float32))(x_pad, W1)
    dinv, g1 = _tc_call(
        _tc_prep_kernel,
        (jax.ShapeDtypeStruct((NPAD, 1), jnp.float32),
         jax.ShapeDtypeStruct((NPAD, HID), jnp.float32)))(deg, h1)

    acc1 = _sc_propagate(g1, src2d, dst2d, zeros_tile)
    g2 = _tc_call(_tc_layer_kernel,
                  jax.ShapeDtypeStruct((NPAD, HID), jnp.float32))(
                      acc1, g1, dinv, b1r, W2)
    acc2 = _sc_propagate(g2, src2d, dst2d, zeros_tile)
    g3 = _tc_call(_tc_layer_kernel,
                  jax.ShapeDtypeStruct((NPAD, HID), jnp.float32))(
                      acc2, g2, dinv, b2r, W3)
    acc3 = _sc_propagate(g3, src2d, dst2d, zeros_tile)
    out = _tc_call(_tc_final_kernel,
                   jax.ShapeDtypeStruct((NPAD, HID), jnp.float32))(
                       acc3, g3, dinv, b3r)
    return out[:N_NODES]


# speedup vs baseline: 15.3632x; 1.0551x over previous
import functools

import jax
import jax.numpy as jnp
from jax import lax
from jax.experimental import pallas as pl
from jax.experimental.pallas import tpu as pltpu
from jax.experimental.pallas import tpu_sc as plsc

N_NODES = 10000
IN_DIM = 128
HID = 64

NPAD = 10240
NC, NS = 2, 16
NW = NC * NS
EPW = 10240
EPAD = NW * EPW
CHUNK = 512
GROUPS = CHUNK // 128
NCHUNK = EPW // CHUNK
ROWS_PER_W = EPW // 128
ROWS_PER_CHUNK = CHUNK // 128
TILE_ROWS = NPAD // NS
DEAD_DST = 10100
DEGW = 16

_sc_mesh = plsc.VectorSubcoreMesh(core_axis_name="c", subcore_axis_name="s")


_HALF = TILE_ROWS // 2


@functools.partial(
    pl.kernel,
    out_type=jax.ShapeDtypeStruct((NC, NPAD, HID), jnp.float32),
    mesh=_sc_mesh,
    scratch_types=[
        pltpu.VMEM((ROWS_PER_W, 128), jnp.int32),
        pltpu.VMEM((ROWS_PER_W, 128), jnp.int32),
        pltpu.VMEM((CHUNK, HID), jnp.float32),
        pltpu.VMEM((CHUNK, HID), jnp.float32),
        pltpu.VMEM_SHARED((NPAD, HID), jnp.float32),
        pltpu.SemaphoreType.DMA,
        pltpu.SemaphoreType.DMA,
        pltpu.SemaphoreType.DMA,
        pltpu.SemaphoreType.DMA,
    ],
    compiler_params=pltpu.CompilerParams(use_tc_tiling_on_sc=False),
)
def _sc_propagate(g_hbm, src_hbm, dst_hbm, zeros_hbm, out_hbm,
                  sidx, didx, rows_a, rows_b, acc_sh,
                  sem_ga, sem_gb, sem_sa, sem_sb):
    c = lax.axis_index("c")
    s = lax.axis_index("s")
    w = s * NC + c

    pltpu.sync_copy(src_hbm.at[pl.ds(w * ROWS_PER_W, ROWS_PER_W)], sidx)
    pltpu.sync_copy(dst_hbm.at[pl.ds(w * ROWS_PER_W, ROWS_PER_W)], didx)

    pltpu.sync_copy(zeros_hbm, rows_a.at[pl.ds(0, _HALF)])
    for h in range(2):
        pltpu.sync_copy(rows_a.at[pl.ds(0, _HALF)],
                        acc_sh.at[pl.ds(s * TILE_ROWS + h * _HALF, _HALF)])
    plsc.subcore_barrier()

    def _drain(sem):
        for _ in range(GROUPS):
            pltpu.make_async_copy(g_hbm.at[pl.ds(0, 128)],
                                  rows_a.at[pl.ds(0, 128)], sem).wait()

    @pl.loop(0, NCHUNK // 2)
    def _pair(i):
        r0 = i * 2 * GROUPS

        @pl.when(i > 0)
        def _():
            _drain(sem_sa)

        ga = [pltpu.async_copy(g_hbm.at[sidx.at[r0 + j]],
                               rows_a.at[pl.ds(j * 128, 128)], sem_ga)
              for j in range(GROUPS)]

        @pl.when(i > 0)
        def _():
            _drain(sem_sb)

        gb = [pltpu.async_copy(g_hbm.at[sidx.at[r0 + GROUPS + j]],
                               rows_b.at[pl.ds(j * 128, 128)], sem_gb)
              for j in range(GROUPS)]
        for d in ga:
            d.wait()
        for j in range(GROUPS):
            pltpu.async_copy(rows_a.at[pl.ds(j * 128, 128)],
                             acc_sh.at[didx.at[r0 + j]], sem_sa, add=True)
        for d in gb:
            d.wait()
        for j in range(GROUPS):
            pltpu.async_copy(rows_b.at[pl.ds(j * 128, 128)],
                             acc_sh.at[didx.at[r0 + GROUPS + j]], sem_sb,
                             add=True)

    _drain(sem_sa)
    _drain(sem_sb)
    plsc.subcore_barrier()
    for h in range(2):
        pltpu.sync_copy(acc_sh.at[pl.ds(s * TILE_ROWS + h * _HALF, _HALF)],
                        rows_a.at[pl.ds(0, _HALF)])
        pltpu.sync_copy(rows_a.at[pl.ds(0, _HALF)],
                        out_hbm.at[c].at[pl.ds(s * TILE_ROWS + h * _HALF,
                                               _HALF)])


@functools.partial(
    pl.kernel,
    out_type=jax.ShapeDtypeStruct((NC, NPAD, DEGW), jnp.float32),
    mesh=_sc_mesh,
    scratch_types=[
        pltpu.VMEM((ROWS_PER_W, 128), jnp.int32),
        pltpu.VMEM((128, DEGW), jnp.float32),
        pltpu.VMEM((TILE_ROWS, DEGW), jnp.float32),
        pltpu.VMEM_SHARED((NPAD, DEGW), jnp.float32),
        pltpu.SemaphoreType.DMA,
    ],
    compiler_params=pltpu.CompilerParams(use_tc_tiling_on_sc=False),
)
def _sc_degree(dst_hbm, ones_hbm, zeros_hbm, out_hbm,
               didx, ones_v, stage, deg_sh, sem):
    c = lax.axis_index("c")
    s = lax.axis_index("s")
    w = s * NC + c

    pltpu.sync_copy(dst_hbm.at[pl.ds(w * ROWS_PER_W, ROWS_PER_W)], didx)
    pltpu.sync_copy(ones_hbm, ones_v)
    pltpu.sync_copy(zeros_hbm, stage)
    pltpu.sync_copy(stage, deg_sh.at[pl.ds(s * TILE_ROWS, TILE_ROWS)])
    plsc.subcore_barrier()

    def _drain_deg():
        for _ in range(2 * GROUPS):
            pltpu.make_async_copy(zeros_hbm.at[pl.ds(0, 128)],
                                  ones_v, sem).wait()

    @pl.loop(0, NCHUNK // 2)
    def _pair(i):
        r0 = i * 2 * GROUPS

        @pl.when(i > 0)
        def _():
            _drain_deg()

        for j in range(2 * GROUPS):
            pltpu.async_copy(ones_v, deg_sh.at[didx.at[r0 + j]], sem,
                             add=True)

    _drain_deg()
    plsc.subcore_barrier()
    pltpu.sync_copy(deg_sh.at[pl.ds(s * TILE_ROWS, TILE_ROWS)], stage)
    pltpu.sync_copy(stage, out_hbm.at[c].at[pl.ds(s * TILE_ROWS, TILE_ROWS)])


def _tc_matmul_kernel(x_ref, w_ref, o_ref):
    o_ref[...] = jnp.dot(x_ref[...], w_ref[...],
                         preferred_element_type=jnp.float32)


def _tc_prep_kernel(deg_ref, h_ref, dinv_ref, g_ref):
    deg = deg_ref[0, :, 0:1] + deg_ref[1, :, 0:1] + 1.0
    dinv = lax.rsqrt(deg)
    dinv_ref[...] = dinv
    g_ref[...] = h_ref[...] * dinv


def _tc_layer_kernel(acc_ref, g_ref, dinv_ref, b_ref, w_ref, o_ref):
    dinv = dinv_ref[...]
    t = dinv * (acc_ref[0] + acc_ref[1] + g_ref[...]) + b_ref[...]
    t = jnp.maximum(t, 0.0)
    o_ref[...] = dinv * jnp.dot(t, w_ref[...],
                                preferred_element_type=jnp.float32)


def _tc_final_kernel(acc_ref, g_ref, dinv_ref, b_ref, o_ref):
    o_ref[...] = (dinv_ref[...] * (acc_ref[0] + acc_ref[1] + g_ref[...])
                  + b_ref[...])


def _tc_call(body, out_shapes):
    return pl.pallas_call(body, out_shape=out_shapes)


def kernel(x, edge_index, W1, b1, W2, b2, W3, b3):
    src = edge_index[0]
    dst = edge_index[1]
    e = src.shape[0]
    fill = EPAD - e
    src2d = jnp.concatenate(
        [src, jnp.zeros((fill,), jnp.int32)]).reshape(EPAD // 128, 128)
    dst2d = jnp.concatenate(
        [dst, jnp.full((fill,), DEAD_DST, jnp.int32)]).reshape(EPAD // 128, 128)
    x_pad = jnp.concatenate(
        [x, jnp.zeros((NPAD - N_NODES, IN_DIM), jnp.float32)])
    zeros_tile = jnp.zeros((_HALF, HID), jnp.float32)
    ones_deg = jnp.ones((128, DEGW), jnp.float32)
    zeros_deg = jnp.zeros((TILE_ROWS, DEGW), jnp.float32)
    b1r = b1.reshape(1, HID)
    b2r = b2.reshape(1, HID)
    b3r = b3.reshape(1, HID)

    deg = _sc_degree(dst2d, ones_deg, zeros_deg)

    h1 = _tc_call(_tc_matmul_kernel,
                  jax.ShapeDtypeStruct((NPAD, HID), jnp.float32))(x_pad, W1)
    dinv, g1 = _tc_call(
        _tc_prep_kernel,
        (jax.ShapeDtypeStruct((NPAD, 1), jnp.float32),
         jax.ShapeDtypeStruct((NPAD, HID), jnp.float32)))(deg, h1)

    acc1 = _sc_propagate(g1, src2d, dst2d, zeros_tile)
    g2 = _tc_call(_tc_layer_kernel,
                  jax.ShapeDtypeStruct((NPAD, HID), jnp.float32))(
                      acc1, g1, dinv, b1r, W2)
    acc2 = _sc_propagate(g2, src2d, dst2d, zeros_tile)
    g3 = _tc_call(_tc_layer_kernel,
                  jax.ShapeDtypeStruct((NPAD, HID), jnp.float32))(
                      acc2, g2, dinv, b2r, W3)
    acc3 = _sc_propagate(g3, src2d, dst2d, zeros_tile)
    out = _tc_call(_tc_final_kernel,
                   jax.ShapeDtypeStruct((NPAD, HID), jnp.float32))(
                       acc3, g3, dinv, b3r)
    return out[:N_NODES]


# speedup vs baseline: 16.0757x; 1.0464x over previous
import functools

import jax
import jax.numpy as jnp
from jax import lax
from jax.experimental import pallas as pl
from jax.experimental.pallas import tpu as pltpu
from jax.experimental.pallas import tpu_sc as plsc

N_NODES = 10000
IN_DIM = 128
HID = 64

NPAD = 10240
NC, NS = 2, 16
NW = NC * NS
EPW = 10240
EPAD = NW * EPW
CHUNK = 512
GROUPS = CHUNK // 128
NCHUNK = EPW // CHUNK
ROWS_PER_W = EPW // 128
ROWS_PER_CHUNK = CHUNK // 128
EPW0, EPW1 = 14336, 6144
NPAIR0 = EPW0 // (2 * CHUNK)
NPAIR1 = EPW1 // (2 * CHUNK)
ROWS_W0 = EPW0 // 128
ROWS_W1 = EPW1 // 128
TILE_ROWS = NPAD // NS
DEAD_DST = 10100
DEGW = 16

_sc_mesh = plsc.VectorSubcoreMesh(core_axis_name="c", subcore_axis_name="s")


_HALF = TILE_ROWS // 2


@functools.partial(
    pl.kernel,
    out_type=jax.ShapeDtypeStruct((NC, NPAD, HID), jnp.float32),
    mesh=_sc_mesh,
    scratch_types=[
        pltpu.VMEM((ROWS_W0, 128), jnp.int32),
        pltpu.VMEM((2 * GROUPS, 128), jnp.int32),
        pltpu.VMEM((CHUNK, HID), jnp.float32),
        pltpu.VMEM((CHUNK, HID), jnp.float32),
        pltpu.VMEM_SHARED((NPAD, HID), jnp.float32),
        pltpu.SemaphoreType.DMA,
        pltpu.SemaphoreType.DMA,
        pltpu.SemaphoreType.DMA,
        pltpu.SemaphoreType.DMA,
        pltpu.SemaphoreType.DMA,
    ],
    compiler_params=pltpu.CompilerParams(use_tc_tiling_on_sc=False),
)
def _sc_propagate(g_hbm, src_hbm, dst_hbm, zeros_hbm, out_hbm,
                  sidx, didx, rows_a, rows_b, acc_sh,
                  sem_ga, sem_gb, sem_sa, sem_sb, sem_i):
    c = lax.axis_index("c")
    s = lax.axis_index("s")
    row_base = jnp.where(c == 0, s * ROWS_W0, NS * ROWS_W0 + s * ROWS_W1)

    @pl.when(c == 0)
    def _():
        pltpu.sync_copy(src_hbm.at[pl.ds(row_base, ROWS_W0)],
                        sidx.at[pl.ds(0, ROWS_W0)])

    @pl.when(c == 1)
    def _():
        pltpu.sync_copy(src_hbm.at[pl.ds(row_base, ROWS_W1)],
                        sidx.at[pl.ds(0, ROWS_W1)])

    npair = jnp.where(c == 0, NPAIR0, NPAIR1)

    pltpu.sync_copy(zeros_hbm, rows_a.at[pl.ds(0, _HALF)])
    for h in range(2):
        pltpu.sync_copy(rows_a.at[pl.ds(0, _HALF)],
                        acc_sh.at[pl.ds(s * TILE_ROWS + h * _HALF, _HALF)])
    plsc.subcore_barrier()

    def _drain(sem):
        for _ in range(GROUPS):
            pltpu.make_async_copy(g_hbm.at[pl.ds(0, 128)],
                                  rows_a.at[pl.ds(0, 128)], sem).wait()

    @pl.loop(0, npair)
    def _pair(i):
        r0 = i * 2 * GROUPS

        @pl.when(i > 0)
        def _():
            _drain(sem_sa)

        ga = [pltpu.async_copy(g_hbm.at[sidx.at[r0 + j]],
                               rows_a.at[pl.ds(j * 128, 128)], sem_ga)
              for j in range(GROUPS)]

        @pl.when(i > 0)
        def _():
            _drain(sem_sb)

        di = pltpu.async_copy(dst_hbm.at[pl.ds(row_base + r0, 2 * GROUPS)],
                              didx, sem_i)
        gb = [pltpu.async_copy(g_hbm.at[sidx.at[r0 + GROUPS + j]],
                               rows_b.at[pl.ds(j * 128, 128)], sem_gb)
              for j in range(GROUPS)]
        for d in ga:
            d.wait()
        di.wait()
        for j in range(GROUPS):
            pltpu.async_copy(rows_a.at[pl.ds(j * 128, 128)],
                             acc_sh.at[didx.at[j]], sem_sa, add=True)
        for d in gb:
            d.wait()
        for j in range(GROUPS):
            pltpu.async_copy(rows_b.at[pl.ds(j * 128, 128)],
                             acc_sh.at[didx.at[GROUPS + j]], sem_sb,
                             add=True)

    _drain(sem_sa)
    _drain(sem_sb)
    plsc.subcore_barrier()
    for h in range(2):
        pltpu.sync_copy(acc_sh.at[pl.ds(s * TILE_ROWS + h * _HALF, _HALF)],
                        rows_a.at[pl.ds(0, _HALF)])
        pltpu.sync_copy(rows_a.at[pl.ds(0, _HALF)],
                        out_hbm.at[c].at[pl.ds(s * TILE_ROWS + h * _HALF,
                                               _HALF)])


@functools.partial(
    pl.kernel,
    out_type=jax.ShapeDtypeStruct((NC, NPAD, DEGW), jnp.float32),
    mesh=_sc_mesh,
    scratch_types=[
        pltpu.VMEM((ROWS_PER_W, 128), jnp.int32),
        pltpu.VMEM((128, DEGW), jnp.float32),
        pltpu.VMEM((TILE_ROWS, DEGW), jnp.float32),
        pltpu.VMEM_SHARED((NPAD, DEGW), jnp.float32),
        pltpu.SemaphoreType.DMA,
    ],
    compiler_params=pltpu.CompilerParams(use_tc_tiling_on_sc=False),
)
def _sc_degree(dst_hbm, ones_hbm, zeros_hbm, out_hbm,
               didx, ones_v, stage, deg_sh, sem):
    c = lax.axis_index("c")
    s = lax.axis_index("s")
    w = s * NC + c

    pltpu.sync_copy(dst_hbm.at[pl.ds(w * ROWS_PER_W, ROWS_PER_W)], didx)
    pltpu.sync_copy(ones_hbm, ones_v)
    pltpu.sync_copy(zeros_hbm, stage)
    pltpu.sync_copy(stage, deg_sh.at[pl.ds(s * TILE_ROWS, TILE_ROWS)])
    plsc.subcore_barrier()

    def _drain_deg():
        for _ in range(2 * GROUPS):
            pltpu.make_async_copy(zeros_hbm.at[pl.ds(0, 128)],
                                  ones_v, sem).wait()

    @pl.loop(0, NCHUNK // 2)
    def _pair(i):
        r0 = i * 2 * GROUPS

        @pl.when(i > 0)
        def _():
            _drain_deg()

        for j in range(2 * GROUPS):
            pltpu.async_copy(ones_v, deg_sh.at[didx.at[r0 + j]], sem,
                             add=True)

    _drain_deg()
    plsc.subcore_barrier()
    pltpu.sync_copy(deg_sh.at[pl.ds(s * TILE_ROWS, TILE_ROWS)], stage)
    pltpu.sync_copy(stage, out_hbm.at[c].at[pl.ds(s * TILE_ROWS, TILE_ROWS)])


def _tc_matmul_kernel(x_ref, w_ref, o_ref):
    o_ref[...] = jnp.dot(x_ref[...], w_ref[...],
                         preferred_element_type=jnp.float32)


def _tc_prep_kernel(deg_ref, h_ref, dinv_ref, g_ref):
    deg = deg_ref[0, :, 0:1] + deg_ref[1, :, 0:1] + 1.0
    dinv = lax.rsqrt(deg)
    dinv_ref[...] = dinv
    g_ref[...] = h_ref[...] * dinv


def _tc_layer_kernel(acc_ref, g_ref, dinv_ref, b_ref, w_ref, o_ref):
    dinv = dinv_ref[...]
    t = dinv * (acc_ref[0] + acc_ref[1] + g_ref[...]) + b_ref[...]
    t = jnp.maximum(t, 0.0)
    o_ref[...] = dinv * jnp.dot(t, w_ref[...],
                                preferred_element_type=jnp.float32)


def _tc_final_kernel(acc_ref, g_ref, dinv_ref, b_ref, o_ref):
    o_ref[...] = (dinv_ref[...] * (acc_ref[0] + acc_ref[1] + g_ref[...])
                  + b_ref[...])


def _tc_call(body, out_shapes):
    return pl.pallas_call(body, out_shape=out_shapes)


def kernel(x, edge_index, W1, b1, W2, b2, W3, b3):
    src = edge_index[0]
    dst = edge_index[1]
    e = src.shape[0]
    fill = EPAD - e
    src2d = jnp.concatenate(
        [src, jnp.zeros((fill,), jnp.int32)]).reshape(EPAD // 128, 128)
    dst2d = jnp.concatenate(
        [dst, jnp.full((fill,), DEAD_DST, jnp.int32)]).reshape(EPAD // 128, 128)
    x_pad = jnp.concatenate(
        [x, jnp.zeros((NPAD - N_NODES, IN_DIM), jnp.float32)])
    zeros_tile = jnp.zeros((_HALF, HID), jnp.float32)
    ones_deg = jnp.ones((128, DEGW), jnp.float32)
    zeros_deg = jnp.zeros((TILE_ROWS, DEGW), jnp.float32)
    b1r = b1.reshape(1, HID)
    b2r = b2.reshape(1, HID)
    b3r = b3.reshape(1, HID)

    deg = _sc_degree(dst2d, ones_deg, zeros_deg)

    h1 = _tc_call(_tc_matmul_kernel,
                  jax.ShapeDtypeStruct((NPAD, HID), jnp.float32))(x_pad, W1)
    dinv, g1 = _tc_call(
        _tc_prep_kernel,
        (jax.ShapeDtypeStruct((NPAD, 1), jnp.float32),
         jax.ShapeDtypeStruct((NPAD, HID), jnp.float32)))(deg, h1)

    acc1 = _sc_propagate(g1, src2d, dst2d, zeros_tile)
    g2 = _tc_call(_tc_layer_kernel,
                  jax.ShapeDtypeStruct((NPAD, HID), jnp.float32))(
                      acc1, g1, dinv, b1r, W2)
    acc2 = _sc_propagate(g2, src2d, dst2d, zeros_tile)
    g3 = _tc_call(_tc_layer_kernel,
                  jax.ShapeDtypeStruct((NPAD, HID), jnp.float32))(
                      acc2, g2, dinv, b2r, W3)
    acc3 = _sc_propagate(g3, src2d, dst2d, zeros_tile)
    out = _tc_call(_tc_final_kernel,
                   jax.ShapeDtypeStruct((NPAD, HID), jnp.float32))(
                       acc3, g3, dinv, b3r)
    return out[:N_NODES]
